# Initial kernel scaffold; baseline (speedup 1.0000x reference)
import jax
import jax.numpy as jnp
from jax.experimental import pallas as pl


def kernel(x, edge_index, W1, b1, W20, b20, W21, b21, W22, b22, W3, b3):
    raise NotImplementedError("write your pallas kernel here")


# speedup vs baseline: 15.4180x; 15.4180x over previous
import functools

import jax
import jax.numpy as jnp
from jax import lax
from jax.experimental import pallas as pl
from jax.experimental.pallas import tpu as pltpu
from jax.experimental.pallas import tpu_sc as plsc

NC = 2
NS = 16
NW = NC * NS
CHUNK = 128
BLK = 512


def _sc_mesh():
  return plsc.VectorSubcoreMesh(
      core_axis_name="c", subcore_axis_name="s",
      num_cores=NC, num_subcores=NS)


@functools.lru_cache(maxsize=None)
def _seg_sum_kernel(n_pad, h, k):
  rps = n_pad // NS

  @functools.partial(
      pl.kernel, mesh=_sc_mesh(),
      compiler_params=pltpu.CompilerParams(use_tc_tiling_on_sc=False),
      out_type=jax.ShapeDtypeStruct((NC * n_pad, h), jnp.float32),
      scratch_types=[
          pltpu.VMEM((k, CHUNK), jnp.int32),
          pltpu.VMEM((k, CHUNK), jnp.int32),
          pltpu.VMEM((CHUNK, h), jnp.float32),
          pltpu.VMEM_SHARED((n_pad, h), jnp.float32),
          pltpu.SemaphoreType.DMA,
      ])
  def seg(hs_hbm, src_hbm, dst_hbm, zero_hbm, out_hbm,
          src_v, dst_v, buf_a, acc, sem_a):
    c = lax.axis_index("c")
    s = lax.axis_index("s")
    w = c * NS + s
    r0 = s * rps
    pltpu.sync_copy(zero_hbm.at[pl.ds(r0, rps)], acc.at[pl.ds(r0, rps)])
    pltpu.sync_copy(src_hbm.at[w], src_v)
    pltpu.sync_copy(dst_hbm.at[w], dst_v)
    plsc.subcore_barrier()

    def body(j, carry):
      pltpu.async_copy(hs_hbm.at[src_v.at[j]], buf_a, sem_a).wait()
      pltpu.sync_copy(buf_a, acc.at[dst_v.at[j]], add=True)
      return carry

    lax.fori_loop(0, k, body, 0)
    plsc.subcore_barrier()
    pltpu.sync_copy(acc.at[pl.ds(r0, rps)],
                    out_hbm.at[pl.ds(c * n_pad + r0, rps)])

  return seg


@functools.lru_cache(maxsize=None)
def _deg_kernel(n_pad, k):
  wdeg = 16
  rps = n_pad // NS

  @functools.partial(
      pl.kernel, mesh=_sc_mesh(),
      compiler_params=pltpu.CompilerParams(use_tc_tiling_on_sc=False),
      out_type=jax.ShapeDtypeStruct((NC * n_pad, wdeg), jnp.float32),
      scratch_types=[
          pltpu.VMEM((k, CHUNK), jnp.int32),
          pltpu.VMEM((CHUNK, wdeg), jnp.float32),
          pltpu.VMEM_SHARED((n_pad, wdeg), jnp.float32),
      ])
  def deg(dst_hbm, ones_hbm, zero_hbm, out_hbm, dst_v, ones_v, acc):
    c = lax.axis_index("c")
    s = lax.axis_index("s")
    w = c * NS + s
    r0 = s * rps
    pltpu.sync_copy(zero_hbm.at[pl.ds(r0, rps)], acc.at[pl.ds(r0, rps)])
    pltpu.sync_copy(dst_hbm.at[w], dst_v)
    pltpu.sync_copy(ones_hbm, ones_v)
    plsc.subcore_barrier()

    def body(j, carry):
      pltpu.sync_copy(ones_v, acc.at[dst_v.at[j]], add=True)
      return carry

    lax.fori_loop(0, k, body, 0)
    plsc.subcore_barrier()
    pltpu.sync_copy(acc.at[pl.ds(r0, rps)],
                    out_hbm.at[pl.ds(c * n_pad + r0, rps)])

  return deg


def _row_specs(n_pad, widths):
  return [pl.BlockSpec((BLK, w), lambda i: (i, 0)) for w in widths]


def _tc_call(body, n_pad, in_specs, out_widths, *args):
  grid = (n_pad // BLK,)
  outs = [jax.ShapeDtypeStruct((n_pad, w), jnp.float32) for w in out_widths]
  res = pl.pallas_call(
      body, grid=grid, in_specs=in_specs,
      out_specs=[pl.BlockSpec((BLK, w), lambda i: (i, 0)) for w in out_widths],
      out_shape=outs)(*args)
  return res


def _shift_spec(n_pad, w):
  off = n_pad // BLK
  return pl.BlockSpec((BLK, w), lambda i: (i + off, 0))


def _tc0_body(sd0, sd1, x, w1, o_hs, o_dis):
  deg = sd0[:, 0:1] + sd1[:, 0:1] + 1.0
  dis = lax.rsqrt(deg)
  z = jnp.dot(x[:], w1[:], preferred_element_type=jnp.float32)
  o_hs[:] = z * dis
  o_dis[:] = jnp.broadcast_to(dis, o_dis.shape)


def _tc_mid_body(s0, s1, hs, disr, w, b, o_hs):
  pre = disr[:] * (s0[:] + s1[:] + hs[:]) + b[:]
  hrelu = jnp.maximum(pre, 0.0)
  o_hs[:] = disr[:] * jnp.dot(hrelu, w[:],
                              preferred_element_type=jnp.float32)


def _tc_last_relu_body(s0, s1, hs, disr, b, o_hs):
  pre = disr[:] * (s0[:] + s1[:] + hs[:]) + b[:]
  o_hs[:] = disr[:] * jnp.maximum(pre, 0.0)


def _tc_final_body(s0, s1, hs, disr, w3p, b3p, o):
  t = disr[:] * (s0[:] + s1[:] + hs[:])
  o[:] = jnp.dot(t, w3p[:], preferred_element_type=jnp.float32) + b3p[:]


def kernel(x, edge_index, W1, b1, W20, b20, W21, b21, W22, b22, W3, b3):
  n, d_in = x.shape
  h = W1.shape[1]
  e = edge_index.shape[1]
  n_pad = -(-(n + NS) // BLK) * BLK
  k = -(-e // (NW * CHUNK))
  k += k % 2
  e_pad = NW * CHUNK * k
  p = e_pad - e

  pad_rows = n_pad - n
  pad_idx = (n + jnp.arange(p, dtype=jnp.int32) % pad_rows)
  srcp = jnp.concatenate([edge_index[0], pad_idx]).reshape(NW, k, CHUNK)
  dstp = jnp.concatenate([edge_index[1], pad_idx]).reshape(NW, k, CHUNK)

  x_pad = jnp.zeros((n_pad, d_in), jnp.float32).at[:n].set(x)
  zeros_h = jnp.zeros((n_pad, h), jnp.float32)
  zeros_16 = jnp.zeros((n_pad, 16), jnp.float32)
  ones_16 = jnp.ones((CHUNK, 16), jnp.float32)
  w3p = jnp.zeros((h, 128), jnp.float32).at[:, :1].set(W3)
  b3p = jnp.zeros((1, 128), jnp.float32).at[0, 0].set(b3[0])
  b1r, b20r, b21r, b22r = (v.reshape(1, h) for v in (b1, b20, b21, b22))

  seg = _seg_sum_kernel(n_pad, h, k)
  deg = _deg_kernel(n_pad, k)

  sd = deg(dstp, ones_16, zeros_16)

  specs0 = [_row_specs(n_pad, [16])[0], _shift_spec(n_pad, 16),
            _row_specs(n_pad, [d_in])[0],
            pl.BlockSpec((d_in, h), lambda i: (0, 0))]
  hs1, disr = _tc_call(_tc0_body, n_pad, specs0, [h, h], sd, sd, x_pad, W1)

  hs = hs1
  for wmat, bvec in ((W20, b1r), (W21, b20r), (W22, b21r)):
    s_part = seg(hs, srcp, dstp, zeros_h)
    sp = [_row_specs(n_pad, [h])[0], _shift_spec(n_pad, h),
          _row_specs(n_pad, [h])[0], _row_specs(n_pad, [h])[0],
          pl.BlockSpec((h, h), lambda i: (0, 0)),
          pl.BlockSpec((1, h), lambda i: (0, 0))]
    (hs,) = _tc_call(_tc_mid_body, n_pad, sp, [h],
                     s_part, s_part, hs, disr, wmat, bvec)

  s_part = seg(hs, srcp, dstp, zeros_h)
  sp = [_row_specs(n_pad, [h])[0], _shift_spec(n_pad, h),
        _row_specs(n_pad, [h])[0], _row_specs(n_pad, [h])[0],
        pl.BlockSpec((1, h), lambda i: (0, 0))]
  (hs5,) = _tc_call(_tc_last_relu_body, n_pad, sp, [h],
                    s_part, s_part, hs, disr, b22r)

  s_part = seg(hs5, srcp, dstp, zeros_h)
  sp = [_row_specs(n_pad, [h])[0], _shift_spec(n_pad, h),
        _row_specs(n_pad, [h])[0], _row_specs(n_pad, [h])[0],
        pl.BlockSpec((h, 128), lambda i: (0, 0)),
        pl.BlockSpec((1, 128), lambda i: (0, 0))]
  (outp,) = _tc_call(_tc_final_body, n_pad, sp, [128],
                     s_part, s_part, hs5, disr, w3p, b3p)
  return outp[:n, :1]


# speedup vs baseline: 18.8759x; 1.2243x over previous
import functools

import jax
import jax.numpy as jnp
from jax import lax
from jax.experimental import pallas as pl
from jax.experimental.pallas import tpu as pltpu
from jax.experimental.pallas import tpu_sc as plsc

NC = 2
NS = 16
NW = NC * NS
CHUNK = 128
BLK = 512


def _sc_mesh():
  return plsc.VectorSubcoreMesh(
      core_axis_name="c", subcore_axis_name="s",
      num_cores=NC, num_subcores=NS)


@functools.lru_cache(maxsize=None)
def _seg_sum_kernel(n_pad, h, k):
  rps = n_pad // NS

  @functools.partial(
      pl.kernel, mesh=_sc_mesh(),
      out_type=jax.ShapeDtypeStruct((NC * n_pad, h), jnp.float32),
      scratch_types=[
          pltpu.VMEM((2, CHUNK), jnp.int32),
          pltpu.VMEM((2, CHUNK), jnp.int32),
          pltpu.VMEM((CHUNK, h), jnp.float32),
          pltpu.VMEM((CHUNK, h), jnp.float32),
          pltpu.VMEM_SHARED((n_pad, h), jnp.float32),
          pltpu.SemaphoreType.DMA,
          pltpu.SemaphoreType.DMA,
          pltpu.SemaphoreType.DMA,
          pltpu.SemaphoreType.DMA,
      ])
  def seg(hs_hbm, sidx_hbm, zero_hbm, out_hbm,
          idx_a, idx_b, buf_a, buf_b, acc, sem_ia, sem_ib, sem_ra, sem_rb):
    c = lax.axis_index("c")
    s = lax.axis_index("s")
    w = c * NS + s
    r0 = s * rps
    cj0 = w * k
    pltpu.sync_copy(zero_hbm.at[pl.ds(r0, rps)], acc.at[pl.ds(r0, rps)])
    plsc.subcore_barrier()

    pltpu.async_copy(sidx_hbm.at[cj0], idx_a, sem_ia)
    pltpu.async_copy(sidx_hbm.at[cj0 + 1], idx_b, sem_ib)
    pltpu.make_async_copy(sidx_hbm.at[cj0], idx_a, sem_ia).wait()
    pltpu.async_copy(hs_hbm.at[idx_a.at[0]], buf_a, sem_ra)

    def pair(jj, carry):
      j = cj0 + 2 * jj
      pltpu.make_async_copy(sidx_hbm.at[j + 1], idx_b, sem_ib).wait()
      pltpu.async_copy(hs_hbm.at[idx_b.at[0]], buf_b, sem_rb)

      pltpu.make_async_copy(hs_hbm.at[idx_a.at[0]], buf_a, sem_ra).wait()
      pltpu.sync_copy(buf_a, acc.at[idx_a.at[1]], add=True)

      @pl.when(2 * jj + 2 < k)
      def _():
        pltpu.async_copy(sidx_hbm.at[j + 2], idx_a, sem_ia)

      pltpu.make_async_copy(hs_hbm.at[idx_b.at[0]], buf_b, sem_rb).wait()
      pltpu.sync_copy(buf_b, acc.at[idx_b.at[1]], add=True)

      @pl.when(2 * jj + 3 < k)
      def _():
        pltpu.async_copy(sidx_hbm.at[j + 3], idx_b, sem_ib)

      @pl.when(2 * jj + 2 < k)
      def _():
        pltpu.make_async_copy(sidx_hbm.at[j + 2], idx_a, sem_ia).wait()
        pltpu.async_copy(hs_hbm.at[idx_a.at[0]], buf_a, sem_ra)

      return carry

    lax.fori_loop(0, k // 2, pair, 0)
    plsc.subcore_barrier()
    pltpu.sync_copy(acc.at[pl.ds(r0, rps)],
                    out_hbm.at[pl.ds(c * n_pad + r0, rps)])

  return seg


@functools.lru_cache(maxsize=None)
def _deg_kernel(n_pad, k):
  wdeg = 16
  rps = n_pad // NS

  @functools.partial(
      pl.kernel, mesh=_sc_mesh(),
      compiler_params=pltpu.CompilerParams(use_tc_tiling_on_sc=False),
      out_type=jax.ShapeDtypeStruct((NC * n_pad, wdeg), jnp.float32),
      scratch_types=[
          pltpu.VMEM((k, CHUNK), jnp.int32),
          pltpu.VMEM((CHUNK, wdeg), jnp.float32),
          pltpu.VMEM_SHARED((n_pad, wdeg), jnp.float32),
      ])
  def deg(dst_hbm, ones_hbm, zero_hbm, out_hbm, dst_v, ones_v, acc):
    c = lax.axis_index("c")
    s = lax.axis_index("s")
    w = c * NS + s
    r0 = s * rps
    pltpu.sync_copy(zero_hbm.at[pl.ds(r0, rps)], acc.at[pl.ds(r0, rps)])
    pltpu.sync_copy(dst_hbm.at[w], dst_v)
    pltpu.sync_copy(ones_hbm, ones_v)
    plsc.subcore_barrier()

    def body(j, carry):
      pltpu.sync_copy(ones_v, acc.at[dst_v.at[j]], add=True)
      return carry

    lax.fori_loop(0, k, body, 0)
    plsc.subcore_barrier()
    pltpu.sync_copy(acc.at[pl.ds(r0, rps)],
                    out_hbm.at[pl.ds(c * n_pad + r0, rps)])

  return deg


def _row_specs(n_pad, widths):
  return [pl.BlockSpec((BLK, w), lambda i: (i, 0)) for w in widths]


def _tc_call(body, n_pad, in_specs, out_widths, *args):
  grid = (n_pad // BLK,)
  outs = [jax.ShapeDtypeStruct((n_pad, w), jnp.float32) for w in out_widths]
  res = pl.pallas_call(
      body, grid=grid, in_specs=in_specs,
      out_specs=[pl.BlockSpec((BLK, w), lambda i: (i, 0)) for w in out_widths],
      out_shape=outs)(*args)
  return res


def _shift_spec(n_pad, w):
  off = n_pad // BLK
  return pl.BlockSpec((BLK, w), lambda i: (i + off, 0))


def _tc0_body(sd0, sd1, x, w1, o_hs, o_dis):
  deg = sd0[:, 0:1] + sd1[:, 0:1] + 1.0
  dis = lax.rsqrt(deg)
  z = jnp.dot(x[:], w1[:], preferred_element_type=jnp.float32)
  o_hs[:] = z * dis
  o_dis[:] = jnp.broadcast_to(dis, o_dis.shape)


def _tc_mid_body(s0, s1, hs, disr, w, b, o_hs):
  pre = disr[:] * (s0[:] + s1[:] + hs[:]) + b[:]
  hrelu = jnp.maximum(pre, 0.0)
  o_hs[:] = disr[:] * jnp.dot(hrelu, w[:],
                              preferred_element_type=jnp.float32)


def _tc_last_relu_body(s0, s1, hs, disr, b, o_hs):
  pre = disr[:] * (s0[:] + s1[:] + hs[:]) + b[:]
  o_hs[:] = disr[:] * jnp.maximum(pre, 0.0)


def _tc_final_body(s0, s1, hs, disr, w3p, b3p, o):
  t = disr[:] * (s0[:] + s1[:] + hs[:])
  o[:] = jnp.dot(t, w3p[:], preferred_element_type=jnp.float32) + b3p[:]


def kernel(x, edge_index, W1, b1, W20, b20, W21, b21, W22, b22, W3, b3):
  n, d_in = x.shape
  h = W1.shape[1]
  e = edge_index.shape[1]
  n_pad = -(-(n + NS) // BLK) * BLK
  k = -(-e // (NW * CHUNK))
  k += k % 2
  e_pad = NW * CHUNK * k
  p = e_pad - e

  pad_rows = n_pad - n
  pad_idx = (n + jnp.arange(p, dtype=jnp.int32) % pad_rows)
  srcp = jnp.concatenate([edge_index[0], pad_idx]).reshape(NW, k, CHUNK)
  dstp = jnp.concatenate([edge_index[1], pad_idx]).reshape(NW, k, CHUNK)
  sidx = jnp.stack([srcp, dstp], axis=2).reshape(NW * k, 2, CHUNK)

  x_pad = jnp.zeros((n_pad, d_in), jnp.float32).at[:n].set(x)
  zeros_h = jnp.zeros((n_pad, h), jnp.float32)
  zeros_16 = jnp.zeros((n_pad, 16), jnp.float32)
  ones_16 = jnp.ones((CHUNK, 16), jnp.float32)
  w3p = jnp.zeros((h, 128), jnp.float32).at[:, :1].set(W3)
  b3p = jnp.zeros((1, 128), jnp.float32).at[0, 0].set(b3[0])
  b1r, b20r, b21r, b22r = (v.reshape(1, h) for v in (b1, b20, b21, b22))

  seg = _seg_sum_kernel(n_pad, h, k)
  deg = _deg_kernel(n_pad, k)

  sd = deg(dstp, ones_16, zeros_16)

  specs0 = [_row_specs(n_pad, [16])[0], _shift_spec(n_pad, 16),
            _row_specs(n_pad, [d_in])[0],
            pl.BlockSpec((d_in, h), lambda i: (0, 0))]
  hs1, disr = _tc_call(_tc0_body, n_pad, specs0, [h, h], sd, sd, x_pad, W1)

  hs = hs1
  for wmat, bvec in ((W20, b1r), (W21, b20r), (W22, b21r)):
    s_part = seg(hs, sidx, zeros_h)
    sp = [_row_specs(n_pad, [h])[0], _shift_spec(n_pad, h),
          _row_specs(n_pad, [h])[0], _row_specs(n_pad, [h])[0],
          pl.BlockSpec((h, h), lambda i: (0, 0)),
          pl.BlockSpec((1, h), lambda i: (0, 0))]
    (hs,) = _tc_call(_tc_mid_body, n_pad, sp, [h],
                     s_part, s_part, hs, disr, wmat, bvec)

  s_part = seg(hs, sidx, zeros_h)
  sp = [_row_specs(n_pad, [h])[0], _shift_spec(n_pad, h),
        _row_specs(n_pad, [h])[0], _row_specs(n_pad, [h])[0],
        pl.BlockSpec((1, h), lambda i: (0, 0))]
  (hs5,) = _tc_call(_tc_last_relu_body, n_pad, sp, [h],
                    s_part, s_part, hs, disr, b22r)

  s_part = seg(hs5, sidx, zeros_h)
  sp = [_row_specs(n_pad, [h])[0], _shift_spec(n_pad, h),
        _row_specs(n_pad, [h])[0], _row_specs(n_pad, [h])[0],
        pl.BlockSpec((h, 128), lambda i: (0, 0)),
        pl.BlockSpec((1, 128), lambda i: (0, 0))]
  (outp,) = _tc_call(_tc_final_body, n_pad, sp, [128],
                     s_part, s_part, hs5, disr, w3p, b3p)
  return outp[:n, :1]


# speedup vs baseline: 21.7055x; 1.1499x over previous
import functools

import jax
import jax.numpy as jnp
from jax import lax
from jax.experimental import pallas as pl
from jax.experimental.pallas import tpu as pltpu
from jax.experimental.pallas import tpu_sc as plsc

NC = 2
NS = 16
NW = NC * NS
CHUNK = 128
BLK = 512


def _sc_mesh():
  return plsc.VectorSubcoreMesh(
      core_axis_name="c", subcore_axis_name="s",
      num_cores=NC, num_subcores=NS)


@functools.lru_cache(maxsize=None)
def _seg_sum_kernel(n_pad, h, k):
  rps = n_pad // NS

  @functools.partial(
      pl.kernel, mesh=_sc_mesh(),
      compiler_params=pltpu.CompilerParams(use_tc_tiling_on_sc=False),
      out_type=jax.ShapeDtypeStruct((NC * n_pad, h), jnp.float32),
      scratch_types=[
          pltpu.VMEM((2, CHUNK), jnp.int32),
          pltpu.VMEM((2, CHUNK), jnp.int32),
          pltpu.VMEM((CHUNK, h), jnp.float32),
          pltpu.VMEM((CHUNK, h), jnp.float32),
          pltpu.VMEM_SHARED((n_pad, h), jnp.float32),
          pltpu.SemaphoreType.DMA,
          pltpu.SemaphoreType.DMA,
          pltpu.SemaphoreType.DMA,
          pltpu.SemaphoreType.DMA,
      ])
  def seg(hs_hbm, sidx_hbm, zero_hbm, out_hbm,
          idx_a, idx_b, buf_a, buf_b, acc, sem_ia, sem_ib, sem_ra, sem_rb):
    c = lax.axis_index("c")
    s = lax.axis_index("s")
    w = c * NS + s
    r0 = s * rps
    cj0 = w * k
    pltpu.sync_copy(zero_hbm.at[pl.ds(r0, rps)], acc.at[pl.ds(r0, rps)])
    plsc.subcore_barrier()

    pltpu.async_copy(sidx_hbm.at[cj0], idx_a, sem_ia)
    pltpu.async_copy(sidx_hbm.at[cj0 + 1], idx_b, sem_ib)
    pltpu.make_async_copy(sidx_hbm.at[cj0], idx_a, sem_ia).wait()
    pltpu.async_copy(hs_hbm.at[idx_a.at[0]], buf_a, sem_ra)

    def pair(jj, carry):
      j = cj0 + 2 * jj
      pltpu.make_async_copy(sidx_hbm.at[j + 1], idx_b, sem_ib).wait()
      pltpu.async_copy(hs_hbm.at[idx_b.at[0]], buf_b, sem_rb)

      pltpu.make_async_copy(hs_hbm.at[idx_a.at[0]], buf_a, sem_ra).wait()
      pltpu.sync_copy(buf_a, acc.at[idx_a.at[1]], add=True)

      @pl.when(2 * jj + 2 < k)
      def _():
        pltpu.async_copy(sidx_hbm.at[j + 2], idx_a, sem_ia)
        pltpu.make_async_copy(sidx_hbm.at[j + 2], idx_a, sem_ia).wait()
        pltpu.async_copy(hs_hbm.at[idx_a.at[0]], buf_a, sem_ra)

      pltpu.make_async_copy(hs_hbm.at[idx_b.at[0]], buf_b, sem_rb).wait()
      pltpu.sync_copy(buf_b, acc.at[idx_b.at[1]], add=True)

      @pl.when(2 * jj + 3 < k)
      def _():
        pltpu.async_copy(sidx_hbm.at[j + 3], idx_b, sem_ib)

      return carry

    lax.fori_loop(0, k // 2, pair, 0)
    plsc.subcore_barrier()
    pltpu.sync_copy(acc.at[pl.ds(r0, rps)],
                    out_hbm.at[pl.ds(c * n_pad + r0, rps)])

  return seg


@functools.lru_cache(maxsize=None)
def _deg_kernel(n_pad, k):
  wdeg = 16
  rps = n_pad // NS

  @functools.partial(
      pl.kernel, mesh=_sc_mesh(),
      compiler_params=pltpu.CompilerParams(use_tc_tiling_on_sc=False),
      out_type=jax.ShapeDtypeStruct((NC * n_pad, wdeg), jnp.float32),
      scratch_types=[
          pltpu.VMEM((k, CHUNK), jnp.int32),
          pltpu.VMEM((CHUNK, wdeg), jnp.float32),
          pltpu.VMEM_SHARED((n_pad, wdeg), jnp.float32),
      ])
  def deg(dst_hbm, ones_hbm, zero_hbm, out_hbm, dst_v, ones_v, acc):
    c = lax.axis_index("c")
    s = lax.axis_index("s")
    w = c * NS + s
    r0 = s * rps
    pltpu.sync_copy(zero_hbm.at[pl.ds(r0, rps)], acc.at[pl.ds(r0, rps)])
    pltpu.sync_copy(dst_hbm.at[w], dst_v)
    pltpu.sync_copy(ones_hbm, ones_v)
    plsc.subcore_barrier()

    def body(j, carry):
      pltpu.sync_copy(ones_v, acc.at[dst_v.at[j]], add=True)
      return carry

    lax.fori_loop(0, k, body, 0)
    plsc.subcore_barrier()
    pltpu.sync_copy(acc.at[pl.ds(r0, rps)],
                    out_hbm.at[pl.ds(c * n_pad + r0, rps)])

  return deg


def _row_specs(n_pad, widths):
  return [pl.BlockSpec((BLK, w), lambda i: (i, 0)) for w in widths]


def _tc_call(body, n_pad, in_specs, out_widths, *args):
  grid = (n_pad // BLK,)
  outs = [jax.ShapeDtypeStruct((n_pad, w), jnp.float32) for w in out_widths]
  res = pl.pallas_call(
      body, grid=grid, in_specs=in_specs,
      out_specs=[pl.BlockSpec((BLK, w), lambda i: (i, 0)) for w in out_widths],
      out_shape=outs)(*args)
  return res


def _shift_spec(n_pad, w):
  off = n_pad // BLK
  return pl.BlockSpec((BLK, w), lambda i: (i + off, 0))


def _tc0_body(x, w1, disr, o_hs):
  z = jnp.dot(x[:], w1[:], preferred_element_type=jnp.float32)
  o_hs[:] = z * disr[:]


def _tc_mid_body(s0, s1, hs, disr, w, b, o_hs):
  pre = disr[:] * (s0[:] + s1[:] + hs[:]) + b[:]
  hrelu = jnp.maximum(pre, 0.0)
  o_hs[:] = disr[:] * jnp.dot(hrelu, w[:],
                              preferred_element_type=jnp.float32)


def _tc4_body(s0, s1, hs, disr, b, w3p, disr16, o_hs):
  pre = disr[:] * (s0[:] + s1[:] + hs[:]) + b[:]
  h4 = jnp.maximum(pre, 0.0)
  o_hs[:] = disr16[:] * jnp.dot(h4, w3p[:],
                                preferred_element_type=jnp.float32)


def _tc_final_body(s0, s1, hs, disr16, b3p, o):
  o[:] = disr16[:] * (s0[:] + s1[:] + hs[:]) + b3p[:]


def kernel(x, edge_index, W1, b1, W20, b20, W21, b21, W22, b22, W3, b3):
  n, d_in = x.shape
  h = W1.shape[1]
  e = edge_index.shape[1]
  n_pad = -(-(n + NS) // BLK) * BLK
  k = -(-e // (NW * CHUNK))
  k += k % 2
  e_pad = NW * CHUNK * k
  p = e_pad - e

  pad_rows = n_pad - n
  pad_idx = (n + jnp.arange(p, dtype=jnp.int32) % pad_rows)
  srcp = jnp.concatenate([edge_index[0], pad_idx]).reshape(NW, k, CHUNK)
  dstp = jnp.concatenate([edge_index[1], pad_idx]).reshape(NW, k, CHUNK)
  sidx = jnp.stack([srcp, dstp], axis=2).reshape(NW * k, 2, CHUNK)

  x_pad = jnp.zeros((n_pad, d_in), jnp.float32).at[:n].set(x)
  zeros_h = jnp.zeros((n_pad, h), jnp.float32)
  zeros_16 = jnp.zeros((n_pad, 16), jnp.float32)
  ones_16 = jnp.ones((CHUNK, 16), jnp.float32)
  w3p = jnp.zeros((h, 16), jnp.float32).at[:, :1].set(W3)
  b3p = jnp.zeros((1, 16), jnp.float32).at[0, 0].set(b3[0])
  b1r, b20r, b21r, b22r = (v.reshape(1, h) for v in (b1, b20, b21, b22))

  seg = _seg_sum_kernel(n_pad, h, k)
  deg = _deg_kernel(n_pad, k)

  sd = deg(dstp, ones_16, zeros_16)

  degv = sd[:n_pad, 0] + sd[n_pad:, 0] + 1.0
  dis = jnp.where(degv > 0, degv ** -0.5, 0.0)
  disr = dis[:, None] * jnp.ones((1, h), jnp.float32)
  disr16 = dis[:, None] * jnp.ones((1, 16), jnp.float32)

  specs0 = [_row_specs(n_pad, [d_in])[0],
            pl.BlockSpec((d_in, h), lambda i: (0, 0)),
            _row_specs(n_pad, [h])[0]]
  (hs1,) = _tc_call(_tc0_body, n_pad, specs0, [h], x_pad, W1, disr)

  hs = hs1
  for wmat, bvec in ((W20, b1r), (W21, b20r), (W22, b21r)):
    s_part = seg(hs, sidx, zeros_h)
    sp = [_row_specs(n_pad, [h])[0], _shift_spec(n_pad, h),
          _row_specs(n_pad, [h])[0], _row_specs(n_pad, [h])[0],
          pl.BlockSpec((h, h), lambda i: (0, 0)),
          pl.BlockSpec((1, h), lambda i: (0, 0))]
    (hs,) = _tc_call(_tc_mid_body, n_pad, sp, [h],
                     s_part, s_part, hs, disr, wmat, bvec)

  s_part = seg(hs, sidx, zeros_h)
  sp = [_row_specs(n_pad, [h])[0], _shift_spec(n_pad, h),
        _row_specs(n_pad, [h])[0], _row_specs(n_pad, [h])[0],
        pl.BlockSpec((1, h), lambda i: (0, 0)),
        pl.BlockSpec((h, 16), lambda i: (0, 0)),
        _row_specs(n_pad, [16])[0]]
  (hs5w,) = _tc_call(_tc4_body, n_pad, sp, [16],
                     s_part, s_part, hs, disr, b22r, w3p, disr16)

  seg16 = _seg_sum_kernel(n_pad, 16, k)
  s_part = seg16(hs5w, sidx, zeros_16)
  sp = [_row_specs(n_pad, [16])[0], _shift_spec(n_pad, 16),
        _row_specs(n_pad, [16])[0], _row_specs(n_pad, [16])[0],
        pl.BlockSpec((1, 16), lambda i: (0, 0))]
  (outp,) = _tc_call(_tc_final_body, n_pad, sp, [16],
                     s_part, s_part, hs5w, disr16, b3p)
  return outp[:n, :1]


# speedup vs baseline: 21.7606x; 1.0025x over previous
import functools

import jax
import jax.numpy as jnp
from jax import lax
from jax.experimental import pallas as pl
from jax.experimental.pallas import tpu as pltpu
from jax.experimental.pallas import tpu_sc as plsc

NC = 2
NS = 16
NW = NC * NS
CHUNK = 128
BLK = 512


def _sc_mesh():
  return plsc.VectorSubcoreMesh(
      core_axis_name="c", subcore_axis_name="s",
      num_cores=NC, num_subcores=NS)


@functools.lru_cache(maxsize=None)
def _seg_sum_kernel(n_pad, h, k):
  rps = n_pad // NS

  @functools.partial(
      pl.kernel, mesh=_sc_mesh(),
      compiler_params=pltpu.CompilerParams(use_tc_tiling_on_sc=False),
      out_type=jax.ShapeDtypeStruct((NC * n_pad, h), jnp.float32),
      scratch_types=[
          pltpu.VMEM((2, CHUNK), jnp.int32),
          pltpu.VMEM((2, CHUNK), jnp.int32),
          pltpu.VMEM((CHUNK, h), jnp.float32),
          pltpu.VMEM((CHUNK, h), jnp.float32),
          pltpu.VMEM_SHARED((n_pad, h), jnp.float32),
          pltpu.SemaphoreType.DMA,
          pltpu.SemaphoreType.DMA,
          pltpu.SemaphoreType.DMA,
          pltpu.SemaphoreType.DMA,
      ])
  def seg(hs_hbm, sidx_hbm, zero_hbm, out_hbm,
          idx_a, idx_b, buf_a, buf_b, acc, sem_ia, sem_ib, sem_ra, sem_rb):
    c = lax.axis_index("c")
    s = lax.axis_index("s")
    w = c * NS + s
    r0 = s * rps
    cj0 = w * k
    pltpu.sync_copy(zero_hbm.at[pl.ds(r0, rps)], acc.at[pl.ds(r0, rps)])
    plsc.subcore_barrier()

    pltpu.async_copy(sidx_hbm.at[cj0], idx_a, sem_ia)
    pltpu.async_copy(sidx_hbm.at[cj0 + 1], idx_b, sem_ib)
    pltpu.make_async_copy(sidx_hbm.at[cj0], idx_a, sem_ia).wait()
    pltpu.async_copy(hs_hbm.at[idx_a.at[0]], buf_a, sem_ra)

    def pair(jj, carry):
      j = cj0 + 2 * jj
      pltpu.make_async_copy(sidx_hbm.at[j + 1], idx_b, sem_ib).wait()
      pltpu.async_copy(hs_hbm.at[idx_b.at[0]], buf_b, sem_rb)

      pltpu.make_async_copy(hs_hbm.at[idx_a.at[0]], buf_a, sem_ra).wait()
      pltpu.sync_copy(buf_a, acc.at[idx_a.at[1]], add=True)

      @pl.when(2 * jj + 2 < k)
      def _():
        pltpu.async_copy(sidx_hbm.at[j + 2], idx_a, sem_ia)
        pltpu.make_async_copy(sidx_hbm.at[j + 2], idx_a, sem_ia).wait()
        pltpu.async_copy(hs_hbm.at[idx_a.at[0]], buf_a, sem_ra)

      pltpu.make_async_copy(hs_hbm.at[idx_b.at[0]], buf_b, sem_rb).wait()
      pltpu.sync_copy(buf_b, acc.at[idx_b.at[1]], add=True)

      @pl.when(2 * jj + 3 < k)
      def _():
        pltpu.async_copy(sidx_hbm.at[j + 3], idx_b, sem_ib)

      return carry

    lax.fori_loop(0, k // 2, pair, 0, unroll=2)
    plsc.subcore_barrier()
    pltpu.sync_copy(acc.at[pl.ds(r0, rps)],
                    out_hbm.at[pl.ds(c * n_pad + r0, rps)])

  return seg


@functools.lru_cache(maxsize=None)
def _deg_kernel(n_pad, k):
  wdeg = 16
  rps = n_pad // NS

  @functools.partial(
      pl.kernel, mesh=_sc_mesh(),
      compiler_params=pltpu.CompilerParams(use_tc_tiling_on_sc=False),
      out_type=jax.ShapeDtypeStruct((NC * n_pad, wdeg), jnp.float32),
      scratch_types=[
          pltpu.VMEM((k, CHUNK), jnp.int32),
          pltpu.VMEM((CHUNK, wdeg), jnp.float32),
          pltpu.VMEM_SHARED((n_pad, wdeg), jnp.float32),
          pltpu.SemaphoreType.DMA,
      ])
  def deg(dst_hbm, ones_hbm, zero_hbm, out_hbm, dst_v, ones_v, acc, sem):
    c = lax.axis_index("c")
    s = lax.axis_index("s")
    w = c * NS + s
    r0 = s * rps
    pltpu.sync_copy(zero_hbm.at[pl.ds(r0, rps)], acc.at[pl.ds(r0, rps)])
    pltpu.sync_copy(dst_hbm.at[w], dst_v)
    pltpu.sync_copy(ones_hbm, ones_v)
    plsc.subcore_barrier()

    def body(j, carry):
      pltpu.async_copy(ones_v, acc.at[dst_v.at[j]], sem, add=True)
      return carry

    lax.fori_loop(0, k, body, 0)

    def drain(j, carry):
      pltpu.make_async_copy(ones_v, acc.at[dst_v.at[0]], sem).wait()
      return carry

    lax.fori_loop(0, k, drain, 0)
    plsc.subcore_barrier()
    pltpu.sync_copy(acc.at[pl.ds(r0, rps)],
                    out_hbm.at[pl.ds(c * n_pad + r0, rps)])

  return deg


def _row_specs(n_pad, widths):
  return [pl.BlockSpec((BLK, w), lambda i: (i, 0)) for w in widths]


def _tc_call(body, n_pad, in_specs, out_widths, *args):
  grid = (n_pad // BLK,)
  outs = [jax.ShapeDtypeStruct((n_pad, w), jnp.float32) for w in out_widths]
  res = pl.pallas_call(
      body, grid=grid, in_specs=in_specs,
      out_specs=[pl.BlockSpec((BLK, w), lambda i: (i, 0)) for w in out_widths],
      out_shape=outs)(*args)
  return res


def _shift_spec(n_pad, w):
  off = n_pad // BLK
  return pl.BlockSpec((BLK, w), lambda i: (i + off, 0))


def _tc0_body(x, w1, disr, o_hs):
  z = jnp.dot(x[:], w1[:], preferred_element_type=jnp.float32)
  o_hs[:] = z * disr[:]


def _tc_mid_body(s0, s1, hs, disr, w, b, o_hs):
  pre = disr[:] * (s0[:] + s1[:] + hs[:]) + b[:]
  hrelu = jnp.maximum(pre, 0.0)
  o_hs[:] = disr[:] * jnp.dot(hrelu, w[:],
                              preferred_element_type=jnp.float32)


def _tc4_body(s0, s1, hs, disr, b, w3p, disr16, o_hs):
  pre = disr[:] * (s0[:] + s1[:] + hs[:]) + b[:]
  h4 = jnp.maximum(pre, 0.0)
  o_hs[:] = disr16[:] * jnp.dot(h4, w3p[:],
                                preferred_element_type=jnp.float32)


def _tc_final_body(s0, s1, hs, disr16, b3p, o):
  o[:] = disr16[:] * (s0[:] + s1[:] + hs[:]) + b3p[:]


def kernel(x, edge_index, W1, b1, W20, b20, W21, b21, W22, b22, W3, b3):
  n, d_in = x.shape
  h = W1.shape[1]
  e = edge_index.shape[1]
  n_pad = -(-(n + NS) // BLK) * BLK
  k = -(-e // (NW * CHUNK))
  k += k % 2
  e_pad = NW * CHUNK * k
  p = e_pad - e

  pad_rows = n_pad - n
  pad_idx = (n + jnp.arange(p, dtype=jnp.int32) % pad_rows)
  srcp = jnp.concatenate([edge_index[0], pad_idx]).reshape(NW, k, CHUNK)
  dstp = jnp.concatenate([edge_index[1], pad_idx]).reshape(NW, k, CHUNK)
  sidx = jnp.stack([srcp, dstp], axis=2).reshape(NW * k, 2, CHUNK)

  x_pad = jnp.zeros((n_pad, d_in), jnp.float32).at[:n].set(x)
  zeros_h = jnp.zeros((n_pad, h), jnp.float32)
  zeros_16 = jnp.zeros((n_pad, 16), jnp.float32)
  ones_16 = jnp.ones((CHUNK, 16), jnp.float32)
  w3p = jnp.zeros((h, 16), jnp.float32).at[:, :1].set(W3)
  b3p = jnp.zeros((1, 16), jnp.float32).at[0, 0].set(b3[0])
  b1r, b20r, b21r, b22r = (v.reshape(1, h) for v in (b1, b20, b21, b22))

  seg = _seg_sum_kernel(n_pad, h, k)
  deg = _deg_kernel(n_pad, k)

  sd = deg(dstp, ones_16, zeros_16)

  degv = sd[:n_pad, 0] + sd[n_pad:, 0] + 1.0
  dis = jnp.where(degv > 0, degv ** -0.5, 0.0)
  disr = dis[:, None] * jnp.ones((1, h), jnp.float32)
  disr16 = dis[:, None] * jnp.ones((1, 16), jnp.float32)

  specs0 = [_row_specs(n_pad, [d_in])[0],
            pl.BlockSpec((d_in, h), lambda i: (0, 0)),
            _row_specs(n_pad, [h])[0]]
  (hs1,) = _tc_call(_tc0_body, n_pad, specs0, [h], x_pad, W1, disr)

  hs = hs1
  for wmat, bvec in ((W20, b1r), (W21, b20r), (W22, b21r)):
    s_part = seg(hs, sidx, zeros_h)
    sp = [_row_specs(n_pad, [h])[0], _shift_spec(n_pad, h),
          _row_specs(n_pad, [h])[0], _row_specs(n_pad, [h])[0],
          pl.BlockSpec((h, h), lambda i: (0, 0)),
          pl.BlockSpec((1, h), lambda i: (0, 0))]
    (hs,) = _tc_call(_tc_mid_body, n_pad, sp, [h],
                     s_part, s_part, hs, disr, wmat, bvec)

  s_part = seg(hs, sidx, zeros_h)
  sp = [_row_specs(n_pad, [h])[0], _shift_spec(n_pad, h),
        _row_specs(n_pad, [h])[0], _row_specs(n_pad, [h])[0],
        pl.BlockSpec((1, h), lambda i: (0, 0)),
        pl.BlockSpec((h, 16), lambda i: (0, 0)),
        _row_specs(n_pad, [16])[0]]
  (hs5w,) = _tc_call(_tc4_body, n_pad, sp, [16],
                     s_part, s_part, hs, disr, b22r, w3p, disr16)

  seg16 = _seg_sum_kernel(n_pad, 16, k)
  s_part = seg16(hs5w, sidx, zeros_16)
  sp = [_row_specs(n_pad, [16])[0], _shift_spec(n_pad, 16),
        _row_specs(n_pad, [16])[0], _row_specs(n_pad, [16])[0],
        pl.BlockSpec((1, 16), lambda i: (0, 0))]
  (outp,) = _tc_call(_tc_final_body, n_pad, sp, [16],
                     s_part, s_part, hs5w, disr16, b3p)
  return outp[:n, :1]
